# baseline (speedup 1.0000x reference)
import functools

import jax
import jax.numpy as jnp
from jax import lax
from jax.experimental import pallas as pl
from jax.experimental.pallas import tpu as pltpu
from jax.experimental.pallas import tpu_sc as plsc

N = 10000
E = 320000
D = 128
G = 64

CH = 100
TPC = 100
RPT = 640
NPAD = 10240

BLK = 1000
NB = N // BLK

_HI = jax.lax.Precision.HIGHEST


NCPAD = 10240


def _make_sc_spmm(with_cnt):
  out_type = [jax.ShapeDtypeStruct((2, NPAD, D), jnp.float32)]
  scratch = [
      pltpu.VMEM((2, CH), jnp.int32),
      pltpu.VMEM((2, CH), jnp.int32),
      pltpu.VMEM((CH, D), jnp.float32),
      pltpu.VMEM((CH, D), jnp.float32),
      pltpu.VMEM_SHARED((NPAD, D), jnp.float32),
      pltpu.SemaphoreType.DMA,
      pltpu.SemaphoreType.DMA,
      pltpu.SemaphoreType.DMA,
      pltpu.SemaphoreType.DMA,
  ]
  if with_cnt:
    out_type.append(jax.ShapeDtypeStruct((2, NCPAD), jnp.float32))
    scratch += [
        pltpu.VMEM((CH,), jnp.float32),
        pltpu.VMEM_SHARED((NCPAD,), jnp.float32),
    ]

  mesh = plsc.VectorSubcoreMesh(core_axis_name="c", subcore_axis_name="s")

  def body(x_hbm, src_hbm, zeros_hbm, ones_hbm, *rest):
    if with_cnt:
      (acc_out, cnt_out, ia, ib, rowa, rowc, acc_sh, sema, semb,
       semia, semib, ones_v, cnt_sh) = rest
    else:
      (acc_out, ia, ib, rowa, rowc, acc_sh, sema, semb, semia, semib) = rest
    c = lax.axis_index("c")
    s = lax.axis_index("s")
    wid = c * 16 + s

    pltpu.sync_copy(zeros_hbm, acc_sh.at[pl.ds(s * RPT, RPT)])
    if with_cnt:
      pltpu.sync_copy(ones_hbm, ones_v)

      def zloop(j, _):
        idx = j * 16 + s
        pltpu.sync_copy(zeros_hbm.at[0], cnt_sh.at[pl.ds(idx * D, D)])
        return 0
      lax.fori_loop(0, NCPAD // (16 * D), zloop, 0)

    plsc.subcore_barrier()

    def _scatter(buf, idxb):
      pltpu.sync_copy(buf, acc_sh.at[idxb.at[1]], add=True)
      if with_cnt:
        pltpu.sync_copy(ones_v, cnt_sh.at[idxb.at[1]], add=True)

    pltpu.async_copy(src_hbm.at[wid, 0], ia, semia)
    pltpu.async_copy(src_hbm.at[wid, 1], ib, semib)
    pltpu.make_async_copy(src_hbm.at[wid, 0], ia, semia).wait()
    pltpu.async_copy(x_hbm.at[ia.at[0]], rowa, sema)

    def chunk2(t, _):
      ja = 2 * t
      pltpu.make_async_copy(src_hbm.at[wid, 0], ib, semib).wait()
      pltpu.async_copy(x_hbm.at[ib.at[0]], rowc, semb)
      pltpu.make_async_copy(x_hbm.at[ia.at[0]], rowa, sema).wait()
      _scatter(rowa, ia)
      pltpu.async_copy(src_hbm.at[wid, ja + 2], ia, semia)
      pltpu.make_async_copy(x_hbm.at[ib.at[0]], rowc, semb).wait()
      _scatter(rowc, ib)
      pltpu.make_async_copy(src_hbm.at[wid, 0], ia, semia).wait()
      pltpu.async_copy(x_hbm.at[ia.at[0]], rowa, sema)
      pltpu.async_copy(src_hbm.at[wid, ja + 3], ib, semib)
      return 0
    lax.fori_loop(0, TPC // 2, chunk2, 0)

    pltpu.make_async_copy(x_hbm.at[ia.at[0]], rowa, sema).wait()
    pltpu.make_async_copy(src_hbm.at[wid, 0], ib, semib).wait()

    plsc.subcore_barrier()
    pltpu.sync_copy(acc_sh.at[pl.ds(s * RPT, RPT)],
                    acc_out.at[c, pl.ds(s * RPT, RPT)])
    if with_cnt:
      @pl.when(s == 0)
      def _():
        pltpu.sync_copy(cnt_sh, cnt_out.at[c])

  return functools.partial(pl.kernel, out_type=out_type, mesh=mesh,
                           scratch_types=scratch)(body)


_sc_spmm_cnt = _make_sc_spmm(True)
_sc_spmm = _make_sc_spmm(False)


def _tc1_body(aggp, cnt0, cnt1, x, batch, wrel, wroot, b,
              h_ref, pool_ref, pool_acc, gcnt_acc):
  i = pl.program_id(0)

  @pl.when(i == 0)
  def _():
    pool_acc[...] = jnp.zeros_like(pool_acc)
    gcnt_acc[...] = jnp.zeros_like(gcnt_acc)

  cnt = cnt0[0, 0, :] + cnt1[0, 0, :]
  inv = 1.0 / jnp.maximum(cnt, 1.0)
  a = aggp[...]
  agg = (a[0] + a[1]) * inv[:, None]
  h = (jnp.dot(agg, wrel[...], precision=_HI)
       + jnp.dot(x[...], wroot[...], precision=_HI) + b[...])
  h = jnp.maximum(h, 0.0)
  h_ref[...] = h

  bt = batch[0, 0, :]
  oh = (bt[None, :] == lax.broadcasted_iota(jnp.int32, (G, BLK), 0)
        ).astype(jnp.float32)
  pool_acc[...] += jnp.dot(oh, h, precision=_HI)
  gcnt_acc[...] += jnp.sum(oh, axis=1, keepdims=True)

  @pl.when(i == NB - 1)
  def _():
    pool_ref[...] = pool_acc[...] / jnp.maximum(gcnt_acc[...], 1.0)


def _tc2_body(aggp, cnt0, cnt1, h, wrel, wroot, b,
              h2_ref, stats_ref, stat_acc):
  i = pl.program_id(0)

  @pl.when(i == 0)
  def _():
    stat_acc[...] = jnp.zeros_like(stat_acc)

  cnt = cnt0[0, 0, :] + cnt1[0, 0, :]
  inv = 1.0 / jnp.maximum(cnt, 1.0)
  a = aggp[...]
  agg = (a[0] + a[1]) * inv[:, None]
  h2 = (jnp.dot(agg, wrel[...], precision=_HI)
        + jnp.dot(h[...], wroot[...], precision=_HI) + b[...])
  h2_ref[...] = h2
  stat_acc[0:1, :] += jnp.sum(h2, axis=0, keepdims=True)
  stat_acc[1:2, :] += jnp.sum(h2 * h2, axis=0, keepdims=True)

  @pl.when(i == NB - 1)
  def _():
    stats_ref[...] = stat_acc[...]


def _tc3_body(h2, stats, gamma, beta, pool1, batch, wl1a, wl1b, bl1, wl2, bl2,
              out_ref, pool_acc, gcnt_acc):
  i = pl.program_id(0)

  @pl.when(i == 0)
  def _():
    pool_acc[...] = jnp.zeros_like(pool_acc)
    gcnt_acc[...] = jnp.zeros_like(gcnt_acc)

  mu = stats[0:1, :] * (1.0 / N)
  ex2 = stats[1:2, :] * (1.0 / N)
  var = ex2 - mu * mu
  rstd = lax.rsqrt(var + 1e-5)
  h2n = (h2[...] - mu) * (rstd * gamma[...]) + beta[...]
  h2n = jnp.maximum(h2n, 0.0)

  bt = batch[0, 0, :]
  oh = (bt[None, :] == lax.broadcasted_iota(jnp.int32, (G, BLK), 0)
        ).astype(jnp.float32)
  pool_acc[...] += jnp.dot(oh, h2n, precision=_HI)
  gcnt_acc[...] += jnp.sum(oh, axis=1, keepdims=True)

  @pl.when(i == NB - 1)
  def _():
    pool2 = pool_acc[...] / jnp.maximum(gcnt_acc[...], 1.0)
    z = (jnp.dot(pool1[...], wl1a[...], precision=_HI)
         + jnp.dot(pool2, wl1b[...], precision=_HI) + bl1[...])
    z = jnp.maximum(z, 0.0)
    out_ref[...] = jnp.dot(z, wl2[...], precision=_HI) + bl2[...]


def _row_spec():
  return pl.BlockSpec((BLK, D), lambda i: (i, 0))


def _full(shape):
  return pl.BlockSpec(shape, lambda i: tuple(0 for _ in shape))


def _vec_spec():
  return pl.BlockSpec((1, 1, BLK), lambda i: (i, 0, 0))


def kernel(x, edge_index, batch, W_rel1, W_root1, b1, W_rel2, W_root2, b2,
           gamma, beta, W_lin1, b_lin1, W_lin2, b_lin2):
  src2 = edge_index[0].reshape(32, TPC, CH)
  dst2 = edge_index[1].reshape(32, TPC, CH)
  idx = jnp.stack([src2, dst2], axis=2)
  idx = jnp.concatenate(
      [idx, jnp.zeros((32, 2, 2, CH), jnp.int32)], axis=1)
  zeros = jnp.zeros((RPT, D), jnp.float32)
  ones = jnp.ones((CH,), jnp.float32)

  aggp1, cntp = _sc_spmm_cnt(x, idx, zeros, ones)

  cnt0 = cntp[0, :N].reshape(NB, 1, BLK)
  cnt1 = cntp[1, :N].reshape(NB, 1, BLK)
  batch3 = batch.reshape(NB, 1, BLK)

  h, pool1 = pl.pallas_call(
      _tc1_body,
      grid=(NB,),
      in_specs=[
          pl.BlockSpec((2, BLK, D), lambda i: (0, i, 0)),
          _vec_spec(), _vec_spec(),
          _row_spec(),
          _vec_spec(),
          _full((D, D)), _full((D, D)), _full((1, D)),
      ],
      out_specs=[_row_spec(), _full((G, D))],
      out_shape=[jax.ShapeDtypeStruct((N, D), jnp.float32),
                 jax.ShapeDtypeStruct((G, D), jnp.float32)],
      scratch_shapes=[pltpu.VMEM((G, D), jnp.float32),
                      pltpu.VMEM((G, 1), jnp.float32)],
  )(aggp1, cnt0, cnt1, x, batch3, W_rel1, W_root1, b1.reshape(1, D))

  aggp2 = _sc_spmm(h, idx, zeros, ones)
  if isinstance(aggp2, (list, tuple)):
    aggp2 = aggp2[0]

  h2, stats = pl.pallas_call(
      _tc2_body,
      grid=(NB,),
      in_specs=[
          pl.BlockSpec((2, BLK, D), lambda i: (0, i, 0)),
          _vec_spec(), _vec_spec(),
          _row_spec(),
          _full((D, D)), _full((D, D)), _full((1, D)),
      ],
      out_specs=[_row_spec(), _full((8, D))],
      out_shape=[jax.ShapeDtypeStruct((N, D), jnp.float32),
                 jax.ShapeDtypeStruct((8, D), jnp.float32)],
      scratch_shapes=[pltpu.VMEM((8, D), jnp.float32)],
  )(aggp2, cnt0, cnt1, h, W_rel2, W_root2, b2.reshape(1, D))

  out = pl.pallas_call(
      _tc3_body,
      grid=(NB,),
      in_specs=[
          _row_spec(),
          _full((8, D)), _full((1, D)), _full((1, D)),
          _full((G, D)),
          _vec_spec(),
          _full((D, D)), _full((D, D)), _full((1, D)),
          _full((D, D)), _full((1, D)),
      ],
      out_specs=_full((G, D)),
      out_shape=jax.ShapeDtypeStruct((G, D), jnp.float32),
      scratch_shapes=[pltpu.VMEM((G, D), jnp.float32),
                      pltpu.VMEM((G, 1), jnp.float32)],
  )(h2, stats, gamma.reshape(1, D), beta.reshape(1, D), pool1, batch3,
    W_lin1[:D], W_lin1[D:], b_lin1.reshape(1, D), W_lin2,
    b_lin2.reshape(1, D))

  return out


# speedup vs baseline: 1.8276x; 1.8276x over previous
import functools

import jax
import jax.numpy as jnp
from jax import lax
from jax.experimental import pallas as pl
from jax.experimental.pallas import tpu as pltpu
from jax.experimental.pallas import tpu_sc as plsc

N = 10000
E = 320000
D = 128
G = 64

CH = 80
TPC = 125
RPT = 640
NPAD = 10240

BLK = 1000
NB = N // BLK

_HI = jax.lax.Precision.HIGHEST


NCPAD = 10240


def _make_sc_spmm(with_cnt):
  out_type = [jax.ShapeDtypeStruct((2, NPAD, D), jnp.float32)]
  scratch = [
      pltpu.VMEM((TPC, CH), jnp.int32),
      pltpu.VMEM((CH,), jnp.int32),
      pltpu.VMEM((CH,), jnp.int32),
      pltpu.VMEM((CH,), jnp.int32),
      pltpu.VMEM((CH,), jnp.int32),
      pltpu.VMEM((CH, D), jnp.float32),
      pltpu.VMEM((CH, D), jnp.float32),
      pltpu.VMEM_SHARED((NPAD, D), jnp.float32),
      pltpu.SemaphoreType.DMA,
      pltpu.SemaphoreType.DMA,
  ]
  if with_cnt:
    out_type.append(jax.ShapeDtypeStruct((2, NCPAD), jnp.float32))
    scratch += [
        pltpu.VMEM((CH,), jnp.float32),
        pltpu.VMEM_SHARED((NCPAD,), jnp.float32),
    ]

  mesh = plsc.VectorSubcoreMesh(core_axis_name="c", subcore_axis_name="s")

  def body(x_hbm, src_hbm, zeros_hbm, ones_hbm, *rest):
    if with_cnt:
      (acc_out, cnt_out, pk_v, sa, da, sb, db, rowa, rowc, acc_sh,
       sema, semb, ones_v, cnt_sh) = rest
    else:
      (acc_out, pk_v, sa, da, sb, db, rowa, rowc, acc_sh, sema, semb) = rest
    c = lax.axis_index("c")
    s = lax.axis_index("s")
    wid = c * 16 + s

    pltpu.sync_copy(zeros_hbm, acc_sh.at[pl.ds(s * RPT, RPT)])
    if with_cnt:
      pltpu.sync_copy(ones_hbm, ones_v)

      def zloop(j, _):
        idx = j * 16 + s
        pltpu.sync_copy(zeros_hbm.at[0], cnt_sh.at[pl.ds(idx * D, D)])
        return 0
      lax.fori_loop(0, NCPAD // (16 * D), zloop, 0)

    pltpu.sync_copy(src_hbm.at[wid], pk_v)
    plsc.subcore_barrier()

    def _scatter(buf, didx):
      pltpu.sync_copy(buf, acc_sh.at[didx], add=True)
      if with_cnt:
        pltpu.sync_copy(ones_v, cnt_sh.at[didx], add=True)

    def _unpack(j, sbuf, dbuf):
      for k in range(CH // 16):
        pk = pk_v[j, pl.ds(k * 16, 16)]
        sbuf[pl.ds(k * 16, 16)] = pk & 0xFFFF
        dbuf[pl.ds(k * 16, 16)] = lax.shift_right_logical(pk, 16)

    _unpack(0, sa, da)
    pltpu.async_copy(x_hbm.at[sa], rowa, sema)

    def chunk2(t, _):
      ja = 2 * t
      _unpack(ja + 1, sb, db)
      pltpu.async_copy(x_hbm.at[sb], rowc, semb)
      pltpu.make_async_copy(x_hbm.at[sa], rowa, sema).wait()
      _scatter(rowa, da)
      _unpack(ja + 2, sa, da)
      pltpu.async_copy(x_hbm.at[sa], rowa, sema)
      pltpu.make_async_copy(x_hbm.at[sb], rowc, semb).wait()
      _scatter(rowc, db)
      return 0
    lax.fori_loop(0, (TPC - 1) // 2, chunk2, 0)

    pltpu.make_async_copy(x_hbm.at[sa], rowa, sema).wait()
    _scatter(rowa, da)

    plsc.subcore_barrier()
    pltpu.sync_copy(acc_sh.at[pl.ds(s * RPT, RPT)],
                    acc_out.at[c, pl.ds(s * RPT, RPT)])
    if with_cnt:
      @pl.when(s == 0)
      def _():
        pltpu.sync_copy(cnt_sh, cnt_out.at[c])

  return functools.partial(pl.kernel, out_type=out_type, mesh=mesh,
                           scratch_types=scratch)(body)


_sc_spmm_cnt = _make_sc_spmm(True)
_sc_spmm = _make_sc_spmm(False)


def _tc1_body(aggp, cnt0, cnt1, x, batch, wrel, wroot, b,
              h_ref, pool_ref, pool_acc, gcnt_acc):
  i = pl.program_id(0)

  @pl.when(i == 0)
  def _():
    pool_acc[...] = jnp.zeros_like(pool_acc)
    gcnt_acc[...] = jnp.zeros_like(gcnt_acc)

  cnt = cnt0[0, 0, :] + cnt1[0, 0, :]
  inv = 1.0 / jnp.maximum(cnt, 1.0)
  a = aggp[...]
  agg = (a[0] + a[1]) * inv[:, None]
  h = (jnp.dot(agg, wrel[...], precision=_HI)
       + jnp.dot(x[...], wroot[...], precision=_HI) + b[...])
  h = jnp.maximum(h, 0.0)
  h_ref[...] = h

  bt = batch[0, 0, :]
  oh = (bt[None, :] == lax.broadcasted_iota(jnp.int32, (G, BLK), 0)
        ).astype(jnp.float32)
  pool_acc[...] += jnp.dot(oh, h, precision=_HI)
  gcnt_acc[...] += jnp.sum(oh, axis=1, keepdims=True)

  @pl.when(i == NB - 1)
  def _():
    pool_ref[...] = pool_acc[...] / jnp.maximum(gcnt_acc[...], 1.0)


def _tc2_body(aggp, cnt0, cnt1, h, wrel, wroot, b,
              h2_ref, stats_ref, stat_acc):
  i = pl.program_id(0)

  @pl.when(i == 0)
  def _():
    stat_acc[...] = jnp.zeros_like(stat_acc)

  cnt = cnt0[0, 0, :] + cnt1[0, 0, :]
  inv = 1.0 / jnp.maximum(cnt, 1.0)
  a = aggp[...]
  agg = (a[0] + a[1]) * inv[:, None]
  h2 = (jnp.dot(agg, wrel[...], precision=_HI)
        + jnp.dot(h[...], wroot[...], precision=_HI) + b[...])
  h2_ref[...] = h2
  stat_acc[0:1, :] += jnp.sum(h2, axis=0, keepdims=True)
  stat_acc[1:2, :] += jnp.sum(h2 * h2, axis=0, keepdims=True)

  @pl.when(i == NB - 1)
  def _():
    stats_ref[...] = stat_acc[...]


def _tc3_body(h2, stats, gamma, beta, pool1, batch, wl1a, wl1b, bl1, wl2, bl2,
              out_ref, pool_acc, gcnt_acc):
  i = pl.program_id(0)

  @pl.when(i == 0)
  def _():
    pool_acc[...] = jnp.zeros_like(pool_acc)
    gcnt_acc[...] = jnp.zeros_like(gcnt_acc)

  mu = stats[0:1, :] * (1.0 / N)
  ex2 = stats[1:2, :] * (1.0 / N)
  var = ex2 - mu * mu
  rstd = lax.rsqrt(var + 1e-5)
  h2n = (h2[...] - mu) * (rstd * gamma[...]) + beta[...]
  h2n = jnp.maximum(h2n, 0.0)

  bt = batch[0, 0, :]
  oh = (bt[None, :] == lax.broadcasted_iota(jnp.int32, (G, BLK), 0)
        ).astype(jnp.float32)
  pool_acc[...] += jnp.dot(oh, h2n, precision=_HI)
  gcnt_acc[...] += jnp.sum(oh, axis=1, keepdims=True)

  @pl.when(i == NB - 1)
  def _():
    pool2 = pool_acc[...] / jnp.maximum(gcnt_acc[...], 1.0)
    z = (jnp.dot(pool1[...], wl1a[...], precision=_HI)
         + jnp.dot(pool2, wl1b[...], precision=_HI) + bl1[...])
    z = jnp.maximum(z, 0.0)
    out_ref[...] = jnp.dot(z, wl2[...], precision=_HI) + bl2[...]


def _row_spec():
  return pl.BlockSpec((BLK, D), lambda i: (i, 0))


def _full(shape):
  return pl.BlockSpec(shape, lambda i: tuple(0 for _ in shape))


def _vec_spec():
  return pl.BlockSpec((1, 1, BLK), lambda i: (i, 0, 0))


def kernel(x, edge_index, batch, W_rel1, W_root1, b1, W_rel2, W_root2, b2,
           gamma, beta, W_lin1, b_lin1, W_lin2, b_lin2):
  src2 = edge_index[0].reshape(32, TPC, CH)
  dst2 = edge_index[1].reshape(32, TPC, CH)
  idx = src2 | (dst2 << 16)
  zeros = jnp.zeros((RPT, D), jnp.float32)
  ones = jnp.ones((CH,), jnp.float32)

  aggp1, cntp = _sc_spmm_cnt(x, idx, zeros, ones)

  cnt0 = cntp[0, :N].reshape(NB, 1, BLK)
  cnt1 = cntp[1, :N].reshape(NB, 1, BLK)
  batch3 = batch.reshape(NB, 1, BLK)

  h, pool1 = pl.pallas_call(
      _tc1_body,
      grid=(NB,),
      in_specs=[
          pl.BlockSpec((2, BLK, D), lambda i: (0, i, 0)),
          _vec_spec(), _vec_spec(),
          _row_spec(),
          _vec_spec(),
          _full((D, D)), _full((D, D)), _full((1, D)),
      ],
      out_specs=[_row_spec(), _full((G, D))],
      out_shape=[jax.ShapeDtypeStruct((N, D), jnp.float32),
                 jax.ShapeDtypeStruct((G, D), jnp.float32)],
      scratch_shapes=[pltpu.VMEM((G, D), jnp.float32),
                      pltpu.VMEM((G, 1), jnp.float32)],
  )(aggp1, cnt0, cnt1, x, batch3, W_rel1, W_root1, b1.reshape(1, D))

  aggp2 = _sc_spmm(h, idx, zeros, ones)
  if isinstance(aggp2, (list, tuple)):
    aggp2 = aggp2[0]

  h2, stats = pl.pallas_call(
      _tc2_body,
      grid=(NB,),
      in_specs=[
          pl.BlockSpec((2, BLK, D), lambda i: (0, i, 0)),
          _vec_spec(), _vec_spec(),
          _row_spec(),
          _full((D, D)), _full((D, D)), _full((1, D)),
      ],
      out_specs=[_row_spec(), _full((8, D))],
      out_shape=[jax.ShapeDtypeStruct((N, D), jnp.float32),
                 jax.ShapeDtypeStruct((8, D), jnp.float32)],
      scratch_shapes=[pltpu.VMEM((8, D), jnp.float32)],
  )(aggp2, cnt0, cnt1, h, W_rel2, W_root2, b2.reshape(1, D))

  out = pl.pallas_call(
      _tc3_body,
      grid=(NB,),
      in_specs=[
          _row_spec(),
          _full((8, D)), _full((1, D)), _full((1, D)),
          _full((G, D)),
          _vec_spec(),
          _full((D, D)), _full((D, D)), _full((1, D)),
          _full((D, D)), _full((1, D)),
      ],
      out_specs=_full((G, D)),
      out_shape=jax.ShapeDtypeStruct((G, D), jnp.float32),
      scratch_shapes=[pltpu.VMEM((G, D), jnp.float32),
                      pltpu.VMEM((G, 1), jnp.float32)],
  )(h2, stats, gamma.reshape(1, D), beta.reshape(1, D), pool1, batch3,
    W_lin1[:D], W_lin1[D:], b_lin1.reshape(1, D), W_lin2,
    b_lin2.reshape(1, D))

  return out
